# baseline (speedup 1.0000x reference)
import functools

import jax
import jax.numpy as jnp
import numpy as np
from jax import lax
from jax.experimental import pallas as pl
from jax.experimental.pallas import tpu as pltpu
from jax.experimental.pallas import tpu_sc as plsc

N = 10000
E = 320000
D = 128
H = 128

NC = 2
NS = 16
NW = NC * NS
NP = 10112
EWP = 10240
EP = EWP * NW
CH = 128
NCH = EWP // CH
ROWS_W = NP // NS
INV_SQRT_H = float(1.0 / np.sqrt(H))

_mesh = plsc.VectorSubcoreMesh(core_axis_name="c", subcore_axis_name="s")
_params = pltpu.CompilerParams(needs_layout_passes=False)


def _worker_id():
    return lax.axis_index("c") * NS + lax.axis_index("s")


def _global_max(maxes_vm):
    rmax = maxes_vm[pl.ds(0, 16)]
    for i in range(1, NW):
        rmax = jnp.maximum(rmax, maxes_vm[pl.ds(i * 16, 16)])
    return jnp.max(rmax)


CHA = 64
NCHA = EWP // CHA


@functools.partial(
    pl.kernel,
    out_type=[
        jax.ShapeDtypeStruct((EP,), jnp.float32),
        jax.ShapeDtypeStruct((NW, 16), jnp.float32),
    ],
    mesh=_mesh,
    compiler_params=_params,
    scratch_types=(
        [pltpu.VMEM((EWP,), jnp.int32)] * 2 +
        [pltpu.VMEM((EWP,), jnp.float32)] +
        [pltpu.VMEM((CHA, D), jnp.float32)] * 8 +
        [pltpu.VMEM((16,), jnp.float32)] +
        [pltpu.SemaphoreType.DMA] * 8
    ),
)
def _sc_logits(q_hbm, k_hbm, src_hbm, dst_hbm, logits_hbm, maxes_hbm,
               srcall, dstall, lall, q0, k0, q1, k1, q2, k2, q3, k3, mxbuf,
               qs0, ks0, qs1, ks1, qs2, ks2, qs3, ks3):
    wid = _worker_id()
    base = wid * EWP
    lanes = lax.iota(jnp.int32, 16)
    inv = jnp.float32(INV_SQRT_H)

    pltpu.sync_copy(src_hbm.at[pl.ds(base, EWP)], srcall)
    pltpu.sync_copy(dst_hbm.at[pl.ds(base, EWP)], dstall)

    qbufs = (q0, q1, q2, q3)
    kbufs = (k0, k1, k2, k3)
    qsems = (qs0, qs1, qs2, qs3)
    ksems = (ks0, ks1, ks2, ks3)

    def issue(t, b):
        pltpu.async_copy(q_hbm.at[dstall.at[pl.ds(t * CHA, CHA)]],
                         qbufs[b], qsems[b])
        pltpu.async_copy(k_hbm.at[srcall.at[pl.ds(t * CHA, CHA)]],
                         kbufs[b], ksems[b])

    def wait(t, b):
        pltpu.make_async_copy(q_hbm.at[dstall.at[pl.ds(t * CHA, CHA)]],
                              qbufs[b], qsems[b]).wait()
        pltpu.make_async_copy(k_hbm.at[srcall.at[pl.ds(t * CHA, CHA)]],
                              kbufs[b], ksems[b]).wait()

    issue(0, 0)
    issue(1, 1)
    issue(2, 2)

    def quad_body(tt, rmax):
        for b in range(4):
            t = tt * 4 + b
            issue(jnp.minimum(t + 3, NCHA - 1), (b + 3) % 4)
            wait(t, b)
            qr = qbufs[b]
            kr = kbufs[b]

            def group_body(g, rmax):
                accs = []
                for j in range(16):
                    r = g * 16 + j
                    a0 = qr[r, pl.ds(0, 16)] * kr[r, pl.ds(0, 16)]
                    a1 = qr[r, pl.ds(16, 16)] * kr[r, pl.ds(16, 16)]
                    for blk in range(2, D // 16, 2):
                        a0 = a0 + qr[r, pl.ds(blk * 16, 16)] * \
                            kr[r, pl.ds(blk * 16, 16)]
                        a1 = a1 + qr[r, pl.ds(blk * 16 + 16, 16)] * \
                            kr[r, pl.ds(blk * 16 + 16, 16)]
                    accs.append(a0 + a1)
                dvs = [jnp.sum(a) for a in accs]
                parts = [jnp.where(lanes == j, dvs[j], 0.0)
                         for j in range(16)]
                while len(parts) > 1:
                    parts = [parts[i] + parts[i + 1]
                             for i in range(0, len(parts), 2)]
                lg = parts[0] * inv
                lall[pl.ds(t * CHA + g * 16, 16)] = lg
                return jnp.maximum(rmax, lg)

            rmax = lax.fori_loop(0, CHA // 16, group_body, rmax)
        return rmax

    rmax = lax.fori_loop(0, NCHA // 4, quad_body,
                         jnp.full((16,), -1e30, jnp.float32))
    wait(NCHA - 1, 0)
    wait(NCHA - 1, 1)
    wait(NCHA - 1, 2)
    pltpu.sync_copy(lall, logits_hbm.at[pl.ds(base, EWP)])
    mxbuf[...] = rmax
    pltpu.sync_copy(mxbuf, maxes_hbm.at[wid])


@functools.partial(
    pl.kernel,
    out_type=jax.ShapeDtypeStruct((NC, NP), jnp.float32),
    mesh=_mesh,
    compiler_params=_params,
    scratch_types=[
        pltpu.VMEM_SHARED((NP,), jnp.float32),
        pltpu.VMEM((NP,), jnp.float32),
        pltpu.VMEM((EWP,), jnp.int32),
        pltpu.VMEM((EWP,), jnp.float32),
        pltpu.VMEM((NW * 16,), jnp.float32),
        pltpu.VMEM((CH,), jnp.int32),
        pltpu.VMEM((CH,), jnp.float32),
    ],
)
def _sc_sums(logits_hbm, dst_hbm, maxes_hbm, sums_hbm,
             s_acc, zvm, dstall, lall, maxes_vm, dsti, echunk):
    cid = lax.axis_index("c")
    sid = lax.axis_index("s")
    wid = cid * NS + sid
    base = wid * EWP

    pltpu.sync_copy(maxes_hbm, maxes_vm)
    gmax = _global_max(maxes_vm)
    pltpu.sync_copy(dst_hbm.at[pl.ds(base, EWP)], dstall)
    pltpu.sync_copy(logits_hbm.at[pl.ds(base, EWP)], lall)

    @pl.when(sid == 0)
    def _zero():
        z16 = jnp.zeros((16,), jnp.float32)

        def zb(i, carry):
            zvm[pl.ds(i * 16, 16)] = z16
            return carry

        lax.fori_loop(0, NP // 16, zb, 0)
        pltpu.sync_copy(zvm, s_acc)

    plsc.subcore_barrier()

    def chunk_body(t, carry):
        for g in range(CH // 16):
            dsti[pl.ds(g * 16, 16)] = dstall[pl.ds(t * CH + g * 16, 16)]
            lv = lall[pl.ds(t * CH + g * 16, 16)]
            echunk[pl.ds(g * 16, 16)] = jnp.exp(lv - gmax)
        pltpu.sync_copy(echunk, s_acc.at[dsti], add=True)
        return carry

    lax.fori_loop(0, NCH, chunk_body, 0)
    plsc.subcore_barrier()

    @pl.when(sid == 0)
    def _dump():
        pltpu.sync_copy(s_acc, zvm)
        pltpu.sync_copy(zvm, sums_hbm.at[cid])


@functools.partial(
    pl.kernel,
    out_type=jax.ShapeDtypeStruct((EP,), jnp.float32),
    mesh=_mesh,
    compiler_params=_params,
    scratch_types=[
        pltpu.VMEM((EWP,), jnp.int32),
        pltpu.VMEM((EWP,), jnp.float32),
        pltpu.VMEM((NP,), jnp.float32),
        pltpu.VMEM((NW * 16,), jnp.float32),
        pltpu.VMEM((CH,), jnp.float32),
    ],
)
def _sc_alpha(logits_hbm, dst_hbm, maxes_hbm, sums_hbm, alpha_hbm,
              dstall, lall, s_full, maxes_vm, sbuf):
    wid = _worker_id()
    base = wid * EWP

    pltpu.sync_copy(maxes_hbm, maxes_vm)
    gmax = _global_max(maxes_vm)
    pltpu.sync_copy(dst_hbm.at[pl.ds(base, EWP)], dstall)
    pltpu.sync_copy(logits_hbm.at[pl.ds(base, EWP)], lall)

    pltpu.sync_copy(sums_hbm.at[0], s_full)
    eps = jnp.float32(1e-16)

    def comb(i, carry):
        pltpu.sync_copy(sums_hbm.at[1, pl.ds(i * CH, CH)], sbuf)
        for g in range(CH // 16):
            sl = pl.ds(i * CH + g * 16, 16)
            s_full[sl] = s_full[sl] + sbuf[pl.ds(g * 16, 16)] + eps
        return carry

    lax.fori_loop(0, NP // CH, comb, 0)

    def seg_body(i, carry):
        sl = pl.ds(i * 16, 16)
        dst16 = dstall[sl]
        sv = plsc.load_gather(s_full, [dst16])
        lall[sl] = jnp.exp(lall[sl] - gmax) / sv
        return carry

    lax.fori_loop(0, EWP // 16, seg_body, 0)
    pltpu.sync_copy(lall, alpha_hbm.at[pl.ds(base, EWP)])


CHC = 80
NCHC = EWP // CHC


@functools.partial(
    pl.kernel,
    out_type=jax.ShapeDtypeStruct((NC, NP, D), jnp.float32),
    mesh=_mesh,
    compiler_params=_params,
    scratch_types=(
        [pltpu.VMEM_SHARED((NP, D), jnp.float32)] +
        [pltpu.VMEM((CHC,), jnp.int32),
         pltpu.VMEM((CHC,), jnp.int32),
         pltpu.VMEM((CHC,), jnp.float32)] * 2 +
        [pltpu.VMEM((CHC, D), jnp.float32)] * 2 +
        [pltpu.VMEM((CHC, D), jnp.float32)] * 2 +
        [pltpu.VMEM((CHC,), jnp.int32)] * 2 +
        [pltpu.SemaphoreType.DMA] * 6
    ),
)
def _sc_aggregate(v_hbm, src_hbm, dst_hbm, alpha_hbm, parts_hbm,
                  out_acc,
                  srci0, dsti0, alf0, srci1, dsti1, alf1,
                  vrows0, vrows1, srows0, srows1, sdst0, sdst1,
                  ms0, ms1, gs0, gs1, ss0, ss1):
    cid = lax.axis_index("c")
    sid = lax.axis_index("s")
    wid = cid * NS + sid
    base = wid * EWP

    srcis = (srci0, srci1)
    dstis = (dsti0, dsti1)
    alfs = (alf0, alf1)
    vbufs = (vrows0, vrows1)
    srows = (srows0, srows1)
    sdsts = (sdst0, sdst1)
    msems = (ms0, ms1)
    gsems = (gs0, gs1)
    ssems = (ss0, ss1)

    def meta_issue(t, b):
        off = base + t * CHC
        pltpu.async_copy(src_hbm.at[pl.ds(off, CHC)], srcis[b], msems[b])
        pltpu.async_copy(dst_hbm.at[pl.ds(off, CHC)], dstis[b], msems[b])
        pltpu.async_copy(alpha_hbm.at[pl.ds(off, CHC)], alfs[b], msems[b])

    def meta_wait(t, b):
        off = base + t * CHC
        pltpu.make_async_copy(src_hbm.at[pl.ds(off, CHC)],
                              srcis[b], msems[b]).wait()
        pltpu.make_async_copy(dst_hbm.at[pl.ds(off, CHC)],
                              dstis[b], msems[b]).wait()
        pltpu.make_async_copy(alpha_hbm.at[pl.ds(off, CHC)],
                              alfs[b], msems[b]).wait()

    def gather_issue(b):
        pltpu.async_copy(v_hbm.at[srcis[b]], vbufs[b], gsems[b])

    def gather_wait(b):
        pltpu.make_async_copy(v_hbm.at[srcis[b]], vbufs[b], gsems[b]).wait()

    def scatter_issue(b):
        pltpu.async_copy(srows[b], out_acc.at[sdsts[b]], ssems[b], add=True)

    def scatter_wait(b):
        pltpu.make_async_copy(srows[b], out_acc.at[sdsts[b]],
                              ssems[b]).wait()

    z16 = jnp.zeros((16,), jnp.float32)
    zi16 = jnp.zeros((16,), jnp.int32)

    @pl.when(sid == 0)
    def _zero():
        def zb(i, carry):
            for blk in range(D // 16):
                vrows0[i, pl.ds(blk * 16, 16)] = z16
            return carry

        lax.fori_loop(0, CHC, zb, 0)

        def zcopy(i, carry):
            pltpu.sync_copy(vrows0.at[pl.ds(0, 64)],
                            out_acc.at[pl.ds(i * 64, 64)])
            return carry

        lax.fori_loop(0, NP // 64, zcopy, 0)

    def zsc(i, carry):
        for blk in range(D // 16):
            srows0[i, pl.ds(blk * 16, 16)] = z16
            srows1[i, pl.ds(blk * 16, 16)] = z16
        return carry

    lax.fori_loop(0, CHC, zsc, 0)
    for g in range(CHC // 16):
        sdst0[pl.ds(g * 16, 16)] = zi16
        sdst1[pl.ds(g * 16, 16)] = zi16

    plsc.subcore_barrier()

    scatter_issue(0)
    scatter_issue(1)

    meta_issue(0, 0)
    meta_wait(0, 0)
    gather_issue(0)
    meta_issue(1, 1)

    def pair_body(tt, carry):
        for b in range(2):
            t = tt * 2 + b
            ob = 1 - b
            gather_wait(b)
            meta_wait(jnp.minimum(t + 1, NCHC - 1), ob)
            gather_issue(ob)
            scatter_wait(b)

            def group_scale(g, carry):
                a16 = alfs[b][pl.ds(g * 16, 16)]
                sdsts[b][pl.ds(g * 16, 16)] = dstis[b][pl.ds(g * 16, 16)]
                for j in range(16):
                    a = a16[j]
                    for blk in range(D // 16):
                        sl = pl.ds(blk * 16, 16)
                        srows[b][g * 16 + j, sl] =                             vbufs[b][g * 16 + j, sl] * a
                return carry

            lax.fori_loop(0, CHC // 16, group_scale, 0)
            scatter_issue(b)
            meta_issue(jnp.minimum(t + 2, NCHC - 1), b)
        return carry

    lax.fori_loop(0, NCHC // 2, pair_body, 0)
    gather_wait(0)
    meta_wait(NCHC - 1, 1)
    scatter_wait(0)
    scatter_wait(1)
    plsc.subcore_barrier()

    pltpu.sync_copy(out_acc.at[pl.ds(sid * ROWS_W, ROWS_W)],
                    parts_hbm.at[cid, pl.ds(sid * ROWS_W, ROWS_W)])


_BLK = 2528


def _proj1_body(x_ref, w_ref, b_ref, q_ref, k_ref, v_ref, s_ref):
    r = jnp.dot(x_ref[...], w_ref[...],
                preferred_element_type=jnp.float32) + b_ref[...]
    q_ref[...] = r[:, 0 * D:1 * D]
    k_ref[...] = r[:, 1 * D:2 * D]
    v_ref[...] = r[:, 2 * D:3 * D]
    s_ref[...] = r[:, 3 * D:4 * D]


def _proj2_body(p0_ref, p1_ref, sk_ref, w_ref, b_ref,
                q_ref, k_ref, v_ref, s_ref):
    h = jnp.maximum(p0_ref[0] + p1_ref[0] + sk_ref[...], 0.0)
    r = jnp.dot(h, w_ref[...], preferred_element_type=jnp.float32) + b_ref[...]
    q_ref[...] = r[:, 0 * D:1 * D]
    k_ref[...] = r[:, 1 * D:2 * D]
    v_ref[...] = r[:, 2 * D:3 * D]
    s_ref[...] = r[:, 3 * D:4 * D]


def _combine_body(p0_ref, p1_ref, sk_ref, o_ref):
    o_ref[...] = p0_ref[0] + p1_ref[0] + sk_ref[...]


def _pad_edges_body(e_ref, s_ref, d_ref):
    s_ref[pl.ds(0, E // 128)] = e_ref[0]
    s_ref[pl.ds(E // 128, (EP - E) // 128)] = jnp.zeros(
        ((EP - E) // 128, 128), jnp.int32)
    d_ref[pl.ds(0, E // 128)] = e_ref[1]
    d_ref[pl.ds(E // 128, (EP - E) // 128)] = jnp.full(
        ((EP - E) // 128, 128), N, jnp.int32)


def _pad_x_body(x_ref, o_ref):
    o_ref[pl.ds(0, N)] = x_ref[...]
    o_ref[pl.ds(N, NP - N)] = jnp.zeros((NP - N, D), jnp.float32)


def _pad_edges(edge_index):
    s2d, d2d = pl.pallas_call(
        _pad_edges_body,
        in_specs=[pl.BlockSpec((2, E // 128, 128), lambda: (0, 0, 0))],
        out_specs=[pl.BlockSpec((EP // 128, 128), lambda: (0, 0))] * 2,
        out_shape=[jax.ShapeDtypeStruct((EP // 128, 128), jnp.int32)] * 2,
    )(edge_index.reshape(2, E // 128, 128))
    return s2d.reshape(EP), d2d.reshape(EP)


def _pad_x(x):
    return pl.pallas_call(
        _pad_x_body,
        in_specs=[pl.BlockSpec((N, D), lambda: (0, 0))],
        out_specs=pl.BlockSpec((NP, D), lambda: (0, 0)),
        out_shape=jax.ShapeDtypeStruct((NP, D), jnp.float32),
    )(x)


def _proj1(x, wcat, bcat):
    return pl.pallas_call(
        _proj1_body,
        grid=(NP // _BLK,),
        in_specs=[
            pl.BlockSpec((_BLK, D), lambda i: (i, 0)),
            pl.BlockSpec((D, 4 * D), lambda i: (0, 0)),
            pl.BlockSpec((1, 4 * D), lambda i: (0, 0)),
        ],
        out_specs=[pl.BlockSpec((_BLK, D), lambda i: (i, 0))] * 4,
        out_shape=[jax.ShapeDtypeStruct((NP, D), jnp.float32)] * 4,
    )(x, wcat, bcat)


def _proj2(parts, sk, wcat, bcat):
    return pl.pallas_call(
        _proj2_body,
        grid=(NP // _BLK,),
        in_specs=[
            pl.BlockSpec((1, _BLK, D), lambda i: (0, i, 0)),
            pl.BlockSpec((1, _BLK, D), lambda i: (1, i, 0)),
            pl.BlockSpec((_BLK, D), lambda i: (i, 0)),
            pl.BlockSpec((D, 4 * D), lambda i: (0, 0)),
            pl.BlockSpec((1, 4 * D), lambda i: (0, 0)),
        ],
        out_specs=[pl.BlockSpec((_BLK, D), lambda i: (i, 0))] * 4,
        out_shape=[jax.ShapeDtypeStruct((NP, D), jnp.float32)] * 4,
    )(parts, parts, sk, wcat, bcat)


def _combine(parts, sk):
    return pl.pallas_call(
        _combine_body,
        grid=(5,),
        in_specs=[
            pl.BlockSpec((1, N // 5, D), lambda i: (0, i, 0)),
            pl.BlockSpec((1, N // 5, D), lambda i: (1, i, 0)),
            pl.BlockSpec((N // 5, D), lambda i: (i, 0)),
        ],
        out_specs=pl.BlockSpec((N // 5, D), lambda i: (i, 0)),
        out_shape=jax.ShapeDtypeStruct((N, D), jnp.float32),
    )(parts, parts, sk)


def _edge_softmax_aggregate(q, k, v, srcp, dstp):
    logits, maxes = _sc_logits(q, k, srcp, dstp)
    maxes = maxes.reshape(-1)
    sums = _sc_sums(logits, dstp, maxes)
    alpha = _sc_alpha(logits, dstp, maxes, sums)
    return _sc_aggregate(v, srcp, dstp, alpha)


def kernel(x, edge_index, Wq1, bq1, Wk1, bk1, Wv1, bv1, Ws1, bs1,
           Wq2, bq2, Wk2, bk2, Wv2, bv2, Ws2, bs2):
    srcp, dstp = _pad_edges(edge_index)
    xp = _pad_x(x)

    w1 = jnp.concatenate([Wq1, Wk1, Wv1, Ws1], axis=1)
    b1 = jnp.concatenate([bq1, bk1, bv1, bs1]).reshape(1, -1)
    w2 = jnp.concatenate([Wq2, Wk2, Wv2, Ws2], axis=1)
    b2 = jnp.concatenate([bq2, bk2, bv2, bs2]).reshape(1, -1)

    q1, k1, v1, sk1 = _proj1(xp, w1, b1)
    parts = _edge_softmax_aggregate(q1, k1, v1, srcp, dstp)
    q2, k2, v2, sk2 = _proj2(parts, sk1, w2, b2)
    parts = _edge_softmax_aggregate(q2, k2, v2, srcp, dstp)
    return _combine(parts, sk2)


# speedup vs baseline: 1.0363x; 1.0363x over previous
import functools

import jax
import jax.numpy as jnp
import numpy as np
from jax import lax
from jax.experimental import pallas as pl
from jax.experimental.pallas import tpu as pltpu
from jax.experimental.pallas import tpu_sc as plsc

N = 10000
E = 320000
D = 128
H = 128

NC = 2
NS = 16
NW = NC * NS
NP = 10112
EWP = 10240
EP = EWP * NW
CH = 128
NCH = EWP // CH
ROWS_W = NP // NS
INV_SQRT_H = float(1.0 / np.sqrt(H))

_mesh = plsc.VectorSubcoreMesh(core_axis_name="c", subcore_axis_name="s")
_params = pltpu.CompilerParams(needs_layout_passes=False)


def _worker_id():
    return lax.axis_index("c") * NS + lax.axis_index("s")


def _global_max(maxes_vm):
    rmax = maxes_vm[pl.ds(0, 16)]
    for i in range(1, NW):
        rmax = jnp.maximum(rmax, maxes_vm[pl.ds(i * 16, 16)])
    return jnp.max(rmax)


CHA = 64
NCHA = EWP // CHA


@functools.partial(
    pl.kernel,
    out_type=[
        jax.ShapeDtypeStruct((EP,), jnp.float32),
        jax.ShapeDtypeStruct((NW, 16), jnp.float32),
        jax.ShapeDtypeStruct((NC, NP), jnp.float32),
    ],
    mesh=_mesh,
    compiler_params=_params,
    scratch_types=(
        [pltpu.VMEM((EWP,), jnp.int32)] * 2 +
        [pltpu.VMEM((EWP,), jnp.float32)] +
        [pltpu.VMEM((CHA, D), jnp.float32)] * 8 +
        [pltpu.VMEM((16,), jnp.float32)] +
        [pltpu.VMEM_SHARED((NP,), jnp.float32)] +
        [pltpu.VMEM_SHARED((NS, 16), jnp.float32)] +
        [pltpu.VMEM((NP,), jnp.float32)] +
        [pltpu.VMEM((NS, 16), jnp.float32)] +
        [pltpu.VMEM((CHA,), jnp.int32)] +
        [pltpu.VMEM((CHA,), jnp.float32)] +
        [pltpu.SemaphoreType.DMA] * 8
    ),
)
def _sc_logsums(q_hbm, k_hbm, src_hbm, dst_hbm,
                logits_hbm, maxes_hbm, sums_hbm,
                srcall, dstall, lall, q0, k0, q1, k1, q2, k2, q3, k3, mxbuf,
                s_acc, mx_sh, zvm, mxv, dsti2, echunk,
                qs0, ks0, qs1, ks1, qs2, ks2, qs3, ks3):
    cid = lax.axis_index("c")
    sid = lax.axis_index("s")
    wid = cid * NS + sid
    base = wid * EWP
    lanes = lax.iota(jnp.int32, 16)
    inv = jnp.float32(INV_SQRT_H)

    @pl.when(sid == 0)
    def _zero():
        z16 = jnp.zeros((16,), jnp.float32)

        def zb(i, carry):
            zvm[pl.ds(i * 16, 16)] = z16
            return carry

        lax.fori_loop(0, NP // 16, zb, 0)
        pltpu.sync_copy(zvm, s_acc)

    pltpu.sync_copy(src_hbm.at[pl.ds(base, EWP)], srcall)
    pltpu.sync_copy(dst_hbm.at[pl.ds(base, EWP)], dstall)

    qbufs = (q0, q1, q2, q3)
    kbufs = (k0, k1, k2, k3)
    qsems = (qs0, qs1, qs2, qs3)
    ksems = (ks0, ks1, ks2, ks3)

    def issue(t, b):
        pltpu.async_copy(q_hbm.at[dstall.at[pl.ds(t * CHA, CHA)]],
                         qbufs[b], qsems[b])
        pltpu.async_copy(k_hbm.at[srcall.at[pl.ds(t * CHA, CHA)]],
                         kbufs[b], ksems[b])

    def wait(t, b):
        pltpu.make_async_copy(q_hbm.at[dstall.at[pl.ds(t * CHA, CHA)]],
                              qbufs[b], qsems[b]).wait()
        pltpu.make_async_copy(k_hbm.at[srcall.at[pl.ds(t * CHA, CHA)]],
                              kbufs[b], ksems[b]).wait()

    issue(0, 0)
    issue(1, 1)
    issue(2, 2)

    def quad_body(tt, rmax):
        for b in range(4):
            t = tt * 4 + b
            issue(jnp.minimum(t + 3, NCHA - 1), (b + 3) % 4)
            wait(t, b)
            qr = qbufs[b]
            kr = kbufs[b]

            def group_body(g, rmax):
                accs = []
                for j in range(16):
                    r = g * 16 + j
                    a0 = qr[r, pl.ds(0, 16)] * kr[r, pl.ds(0, 16)]
                    a1 = qr[r, pl.ds(16, 16)] * kr[r, pl.ds(16, 16)]
                    for blk in range(2, D // 16, 2):
                        a0 = a0 + qr[r, pl.ds(blk * 16, 16)] * \
                            kr[r, pl.ds(blk * 16, 16)]
                        a1 = a1 + qr[r, pl.ds(blk * 16 + 16, 16)] * \
                            kr[r, pl.ds(blk * 16 + 16, 16)]
                    accs.append(a0 + a1)
                dvs = [jnp.sum(a) for a in accs]
                parts = [jnp.where(lanes == j, dvs[j], 0.0)
                         for j in range(16)]
                while len(parts) > 1:
                    parts = [parts[i] + parts[i + 1]
                             for i in range(0, len(parts), 2)]
                lg = parts[0] * inv
                lall[pl.ds(t * CHA + g * 16, 16)] = lg
                return jnp.maximum(rmax, lg)

            rmax = lax.fori_loop(0, CHA // 16, group_body, rmax)
        return rmax

    rmax = lax.fori_loop(0, NCHA // 4, quad_body,
                         jnp.full((16,), -1e30, jnp.float32))
    wait(NCHA - 1, 0)
    wait(NCHA - 1, 1)
    wait(NCHA - 1, 2)
    pltpu.sync_copy(lall, logits_hbm.at[pl.ds(base, EWP)])
    mxbuf[...] = rmax
    pltpu.sync_copy(mxbuf, maxes_hbm.at[wid])
    pltpu.sync_copy(mxbuf, mx_sh.at[sid])

    plsc.subcore_barrier()

    pltpu.sync_copy(mx_sh, mxv)
    mc = mxv[0, pl.ds(0, 16)]
    for i in range(1, NS):
        mc = jnp.maximum(mc, mxv[i, pl.ds(0, 16)])
    mcs = jnp.max(mc)

    def chunk_body(t, carry):
        for g in range(CHA // 16):
            sl = pl.ds(g * 16, 16)
            dsti2[sl] = dstall[pl.ds(t * CHA + g * 16, 16)]
            echunk[sl] = jnp.exp(lall[pl.ds(t * CHA + g * 16, 16)] - mcs)
        pltpu.sync_copy(echunk, s_acc.at[dsti2], add=True)
        return carry

    lax.fori_loop(0, NCHA, chunk_body, 0)
    plsc.subcore_barrier()

    @pl.when(sid == 0)
    def _dump():
        pltpu.sync_copy(s_acc, zvm)
        pltpu.sync_copy(zvm, sums_hbm.at[cid])


@functools.partial(
    pl.kernel,
    out_type=jax.ShapeDtypeStruct((EP,), jnp.float32),
    mesh=_mesh,
    compiler_params=_params,
    scratch_types=[
        pltpu.VMEM((EWP,), jnp.int32),
        pltpu.VMEM((EWP,), jnp.float32),
        pltpu.VMEM((NP,), jnp.float32),
        pltpu.VMEM((NW * 16,), jnp.float32),
        pltpu.VMEM((CH,), jnp.float32),
    ],
)
def _sc_alpha(logits_hbm, dst_hbm, maxes_hbm, sums_hbm, alpha_hbm,
              dstall, lall, s_full, maxes_vm, sbuf):
    wid = _worker_id()
    base = wid * EWP

    pltpu.sync_copy(maxes_hbm, maxes_vm)
    m0v = maxes_vm[pl.ds(0, 16)]
    m1v = maxes_vm[pl.ds(NS * 16, 16)]
    for i in range(1, NS):
        m0v = jnp.maximum(m0v, maxes_vm[pl.ds(i * 16, 16)])
        m1v = jnp.maximum(m1v, maxes_vm[pl.ds((NS + i) * 16, 16)])
    m0 = jnp.max(m0v)
    m1 = jnp.max(m1v)
    gmax = jnp.maximum(m0, m1)
    e0 = jnp.exp(jnp.full((16,), m0 - gmax, jnp.float32))
    e1 = jnp.exp(jnp.full((16,), m1 - gmax, jnp.float32))
    pltpu.sync_copy(dst_hbm.at[pl.ds(base, EWP)], dstall)
    pltpu.sync_copy(logits_hbm.at[pl.ds(base, EWP)], lall)

    pltpu.sync_copy(sums_hbm.at[0], s_full)
    eps = jnp.float32(1e-16)

    def comb(i, carry):
        pltpu.sync_copy(sums_hbm.at[1, pl.ds(i * CH, CH)], sbuf)
        for g in range(CH // 16):
            sl = pl.ds(i * CH + g * 16, 16)
            s_full[sl] = s_full[sl] * e0 + sbuf[pl.ds(g * 16, 16)] * e1 + eps
        return carry

    lax.fori_loop(0, NP // CH, comb, 0)

    def seg_body(i, carry):
        sl = pl.ds(i * 16, 16)
        dst16 = dstall[sl]
        sv = plsc.load_gather(s_full, [dst16])
        lall[sl] = jnp.exp(lall[sl] - gmax) / sv
        return carry

    lax.fori_loop(0, EWP // 16, seg_body, 0)
    pltpu.sync_copy(lall, alpha_hbm.at[pl.ds(base, EWP)])


CHC = 80
NCHC = EWP // CHC


@functools.partial(
    pl.kernel,
    out_type=jax.ShapeDtypeStruct((NC, NP, D), jnp.float32),
    mesh=_mesh,
    compiler_params=_params,
    scratch_types=(
        [pltpu.VMEM_SHARED((NP, D), jnp.float32)] +
        [pltpu.VMEM((CHC,), jnp.int32),
         pltpu.VMEM((CHC,), jnp.int32),
         pltpu.VMEM((CHC,), jnp.float32)] * 2 +
        [pltpu.VMEM((CHC, D), jnp.float32)] * 2 +
        [pltpu.VMEM((CHC, D), jnp.float32)] * 2 +
        [pltpu.VMEM((CHC,), jnp.int32)] * 2 +
        [pltpu.SemaphoreType.DMA] * 6
    ),
)
def _sc_aggregate(v_hbm, src_hbm, dst_hbm, alpha_hbm, parts_hbm,
                  out_acc,
                  srci0, dsti0, alf0, srci1, dsti1, alf1,
                  vrows0, vrows1, srows0, srows1, sdst0, sdst1,
                  ms0, ms1, gs0, gs1, ss0, ss1):
    cid = lax.axis_index("c")
    sid = lax.axis_index("s")
    wid = cid * NS + sid
    base = wid * EWP

    srcis = (srci0, srci1)
    dstis = (dsti0, dsti1)
    alfs = (alf0, alf1)
    vbufs = (vrows0, vrows1)
    srows = (srows0, srows1)
    sdsts = (sdst0, sdst1)
    msems = (ms0, ms1)
    gsems = (gs0, gs1)
    ssems = (ss0, ss1)

    def meta_issue(t, b):
        off = base + t * CHC
        pltpu.async_copy(src_hbm.at[pl.ds(off, CHC)], srcis[b], msems[b])
        pltpu.async_copy(dst_hbm.at[pl.ds(off, CHC)], dstis[b], msems[b])
        pltpu.async_copy(alpha_hbm.at[pl.ds(off, CHC)], alfs[b], msems[b])

    def meta_wait(t, b):
        off = base + t * CHC
        pltpu.make_async_copy(src_hbm.at[pl.ds(off, CHC)],
                              srcis[b], msems[b]).wait()
        pltpu.make_async_copy(dst_hbm.at[pl.ds(off, CHC)],
                              dstis[b], msems[b]).wait()
        pltpu.make_async_copy(alpha_hbm.at[pl.ds(off, CHC)],
                              alfs[b], msems[b]).wait()

    def gather_issue(b):
        pltpu.async_copy(v_hbm.at[srcis[b]], vbufs[b], gsems[b])

    def gather_wait(b):
        pltpu.make_async_copy(v_hbm.at[srcis[b]], vbufs[b], gsems[b]).wait()

    def scatter_issue(b):
        pltpu.async_copy(srows[b], out_acc.at[sdsts[b]], ssems[b], add=True)

    def scatter_wait(b):
        pltpu.make_async_copy(srows[b], out_acc.at[sdsts[b]],
                              ssems[b]).wait()

    z16 = jnp.zeros((16,), jnp.float32)
    zi16 = jnp.zeros((16,), jnp.int32)

    @pl.when(sid == 0)
    def _zero():
        def zb(i, carry):
            for blk in range(D // 16):
                vrows0[i, pl.ds(blk * 16, 16)] = z16
            return carry

        lax.fori_loop(0, CHC, zb, 0)

        def zcopy(i, carry):
            pltpu.sync_copy(vrows0.at[pl.ds(0, 64)],
                            out_acc.at[pl.ds(i * 64, 64)])
            return carry

        lax.fori_loop(0, NP // 64, zcopy, 0)

    def zsc(i, carry):
        for blk in range(D // 16):
            srows0[i, pl.ds(blk * 16, 16)] = z16
            srows1[i, pl.ds(blk * 16, 16)] = z16
        return carry

    lax.fori_loop(0, CHC, zsc, 0)
    for g in range(CHC // 16):
        sdst0[pl.ds(g * 16, 16)] = zi16
        sdst1[pl.ds(g * 16, 16)] = zi16

    plsc.subcore_barrier()

    scatter_issue(0)
    scatter_issue(1)

    meta_issue(0, 0)
    meta_wait(0, 0)
    gather_issue(0)
    meta_issue(1, 1)

    def pair_body(tt, carry):
        for b in range(2):
            t = tt * 2 + b
            ob = 1 - b
            gather_wait(b)
            meta_wait(jnp.minimum(t + 1, NCHC - 1), ob)
            gather_issue(ob)
            scatter_wait(b)

            def group_scale(g, carry):
                a16 = alfs[b][pl.ds(g * 16, 16)]
                sdsts[b][pl.ds(g * 16, 16)] = dstis[b][pl.ds(g * 16, 16)]
                for j in range(16):
                    a = a16[j]
                    for blk in range(D // 16):
                        sl = pl.ds(blk * 16, 16)
                        srows[b][g * 16 + j, sl] =                             vbufs[b][g * 16 + j, sl] * a
                return carry

            lax.fori_loop(0, CHC // 16, group_scale, 0)
            scatter_issue(b)
            meta_issue(jnp.minimum(t + 2, NCHC - 1), b)
        return carry

    lax.fori_loop(0, NCHC // 2, pair_body, 0)
    gather_wait(0)
    meta_wait(NCHC - 1, 1)
    scatter_wait(0)
    scatter_wait(1)
    plsc.subcore_barrier()

    pltpu.sync_copy(out_acc.at[pl.ds(sid * ROWS_W, ROWS_W)],
                    parts_hbm.at[cid, pl.ds(sid * ROWS_W, ROWS_W)])


_BLK = 2528


def _proj1_body(x_ref, w_ref, b_ref, q_ref, k_ref, v_ref, s_ref):
    r = jnp.dot(x_ref[...], w_ref[...],
                preferred_element_type=jnp.float32) + b_ref[...]
    q_ref[...] = r[:, 0 * D:1 * D]
    k_ref[...] = r[:, 1 * D:2 * D]
    v_ref[...] = r[:, 2 * D:3 * D]
    s_ref[...] = r[:, 3 * D:4 * D]


def _proj2_body(p0_ref, p1_ref, sk_ref, w_ref, b_ref,
                q_ref, k_ref, v_ref, s_ref):
    h = jnp.maximum(p0_ref[0] + p1_ref[0] + sk_ref[...], 0.0)
    r = jnp.dot(h, w_ref[...], preferred_element_type=jnp.float32) + b_ref[...]
    q_ref[...] = r[:, 0 * D:1 * D]
    k_ref[...] = r[:, 1 * D:2 * D]
    v_ref[...] = r[:, 2 * D:3 * D]
    s_ref[...] = r[:, 3 * D:4 * D]


def _combine_body(p0_ref, p1_ref, sk_ref, o_ref):
    o_ref[...] = p0_ref[0] + p1_ref[0] + sk_ref[...]


def _pad_edges_body(e_ref, s_ref, d_ref):
    s_ref[pl.ds(0, E // 128)] = e_ref[0]
    s_ref[pl.ds(E // 128, (EP - E) // 128)] = jnp.zeros(
        ((EP - E) // 128, 128), jnp.int32)
    d_ref[pl.ds(0, E // 128)] = e_ref[1]
    d_ref[pl.ds(E // 128, (EP - E) // 128)] = jnp.full(
        ((EP - E) // 128, 128), N, jnp.int32)


def _pad_x_body(x_ref, o_ref):
    o_ref[pl.ds(0, N)] = x_ref[...]
    o_ref[pl.ds(N, NP - N)] = jnp.zeros((NP - N, D), jnp.float32)


def _pad_edges(edge_index):
    s2d, d2d = pl.pallas_call(
        _pad_edges_body,
        in_specs=[pl.BlockSpec((2, E // 128, 128), lambda: (0, 0, 0))],
        out_specs=[pl.BlockSpec((EP // 128, 128), lambda: (0, 0))] * 2,
        out_shape=[jax.ShapeDtypeStruct((EP // 128, 128), jnp.int32)] * 2,
    )(edge_index.reshape(2, E // 128, 128))
    return s2d.reshape(EP), d2d.reshape(EP)


def _pad_x(x):
    return pl.pallas_call(
        _pad_x_body,
        in_specs=[pl.BlockSpec((N, D), lambda: (0, 0))],
        out_specs=pl.BlockSpec((NP, D), lambda: (0, 0)),
        out_shape=jax.ShapeDtypeStruct((NP, D), jnp.float32),
    )(x)


def _proj1(x, wcat, bcat):
    return pl.pallas_call(
        _proj1_body,
        grid=(NP // _BLK,),
        in_specs=[
            pl.BlockSpec((_BLK, D), lambda i: (i, 0)),
            pl.BlockSpec((D, 4 * D), lambda i: (0, 0)),
            pl.BlockSpec((1, 4 * D), lambda i: (0, 0)),
        ],
        out_specs=[pl.BlockSpec((_BLK, D), lambda i: (i, 0))] * 4,
        out_shape=[jax.ShapeDtypeStruct((NP, D), jnp.float32)] * 4,
    )(x, wcat, bcat)


def _proj2(parts, sk, wcat, bcat):
    return pl.pallas_call(
        _proj2_body,
        grid=(NP // _BLK,),
        in_specs=[
            pl.BlockSpec((1, _BLK, D), lambda i: (0, i, 0)),
            pl.BlockSpec((1, _BLK, D), lambda i: (1, i, 0)),
            pl.BlockSpec((_BLK, D), lambda i: (i, 0)),
            pl.BlockSpec((D, 4 * D), lambda i: (0, 0)),
            pl.BlockSpec((1, 4 * D), lambda i: (0, 0)),
        ],
        out_specs=[pl.BlockSpec((_BLK, D), lambda i: (i, 0))] * 4,
        out_shape=[jax.ShapeDtypeStruct((NP, D), jnp.float32)] * 4,
    )(parts, parts, sk, wcat, bcat)


def _combine(parts, sk):
    return pl.pallas_call(
        _combine_body,
        grid=(5,),
        in_specs=[
            pl.BlockSpec((1, N // 5, D), lambda i: (0, i, 0)),
            pl.BlockSpec((1, N // 5, D), lambda i: (1, i, 0)),
            pl.BlockSpec((N // 5, D), lambda i: (i, 0)),
        ],
        out_specs=pl.BlockSpec((N // 5, D), lambda i: (i, 0)),
        out_shape=jax.ShapeDtypeStruct((N, D), jnp.float32),
    )(parts, parts, sk)


def _edge_softmax_aggregate(q, k, v, srcp, dstp):
    logits, maxes, sums = _sc_logsums(q, k, srcp, dstp)
    alpha = _sc_alpha(logits, dstp, maxes.reshape(-1), sums)
    return _sc_aggregate(v, srcp, dstp, alpha)


def kernel(x, edge_index, Wq1, bq1, Wk1, bk1, Wv1, bv1, Ws1, bs1,
           Wq2, bq2, Wk2, bk2, Wv2, bv2, Ws2, bs2):
    srcp, dstp = _pad_edges(edge_index)
    xp = _pad_x(x)

    w1 = jnp.concatenate([Wq1, Wk1, Wv1, Ws1], axis=1)
    b1 = jnp.concatenate([bq1, bk1, bv1, bs1]).reshape(1, -1)
    w2 = jnp.concatenate([Wq2, Wk2, Wv2, Ws2], axis=1)
    b2 = jnp.concatenate([bq2, bk2, bv2, bs2]).reshape(1, -1)

    q1, k1, v1, sk1 = _proj1(xp, w1, b1)
    parts = _edge_softmax_aggregate(q1, k1, v1, srcp, dstp)
    q2, k2, v2, sk2 = _proj2(parts, sk1, w2, b2)
    parts = _edge_softmax_aggregate(q2, k2, v2, srcp, dstp)
    return _combine(parts, sk2)


# speedup vs baseline: 1.0822x; 1.0442x over previous
import functools

import jax
import jax.numpy as jnp
import numpy as np
from jax import lax
from jax.experimental import pallas as pl
from jax.experimental.pallas import tpu as pltpu
from jax.experimental.pallas import tpu_sc as plsc

N = 10000
E = 320000
D = 128
H = 128

NC = 2
NS = 16
NW = NC * NS
NP = 10112
EWP = 10240
EP = EWP * NW
CH = 128
NCH = EWP // CH
ROWS_W = NP // NS
INV_SQRT_H = float(1.0 / np.sqrt(H))

_mesh = plsc.VectorSubcoreMesh(core_axis_name="c", subcore_axis_name="s")
_params = pltpu.CompilerParams(needs_layout_passes=False)


def _worker_id():
    return lax.axis_index("c") * NS + lax.axis_index("s")


def _global_max(maxes_vm):
    rmax = maxes_vm[pl.ds(0, 16)]
    for i in range(1, NW):
        rmax = jnp.maximum(rmax, maxes_vm[pl.ds(i * 16, 16)])
    return jnp.max(rmax)


CHA = 64
NCHA = EWP // CHA


@functools.partial(
    pl.kernel,
    out_type=[
        jax.ShapeDtypeStruct((EP,), jnp.float32),
        jax.ShapeDtypeStruct((NW, 16), jnp.float32),
        jax.ShapeDtypeStruct((NC, NP), jnp.float32),
    ],
    mesh=_mesh,
    compiler_params=_params,
    scratch_types=(
        [pltpu.VMEM((EWP,), jnp.int32)] * 2 +
        [pltpu.VMEM((EWP,), jnp.float32)] +
        [pltpu.VMEM((CHA, D), jnp.float32)] * 8 +
        [pltpu.VMEM((16,), jnp.float32)] +
        [pltpu.VMEM_SHARED((NP,), jnp.float32)] +
        [pltpu.VMEM_SHARED((NS, 16), jnp.float32)] +
        [pltpu.VMEM((NP,), jnp.float32)] +
        [pltpu.VMEM((NS, 16), jnp.float32)] +
        [pltpu.VMEM((CHA,), jnp.int32)] +
        [pltpu.VMEM((CHA,), jnp.float32)] +
        [pltpu.SemaphoreType.DMA] * 8
    ),
)
def _sc_logsums(q_hbm, k_hbm, src_hbm, dst_hbm,
                logits_hbm, maxes_hbm, sums_hbm,
                srcall, dstall, lall, q0, k0, q1, k1, q2, k2, q3, k3, mxbuf,
                s_acc, mx_sh, zvm, mxv, dsti2, echunk,
                qs0, ks0, qs1, ks1, qs2, ks2, qs3, ks3):
    cid = lax.axis_index("c")
    sid = lax.axis_index("s")
    wid = cid * NS + sid
    base = wid * EWP
    lanes = lax.iota(jnp.int32, 16)
    inv = jnp.float32(INV_SQRT_H)

    @pl.when(sid == 0)
    def _zero():
        z16 = jnp.zeros((16,), jnp.float32)

        def zb(i, carry):
            zvm[pl.ds(i * 16, 16)] = z16
            return carry

        lax.fori_loop(0, NP // 16, zb, 0)
        pltpu.sync_copy(zvm, s_acc)

    pltpu.sync_copy(src_hbm.at[pl.ds(base, EWP)], srcall)
    pltpu.sync_copy(dst_hbm.at[pl.ds(base, EWP)], dstall)

    qbufs = (q0, q1, q2, q3)
    kbufs = (k0, k1, k2, k3)
    qsems = (qs0, qs1, qs2, qs3)
    ksems = (ks0, ks1, ks2, ks3)

    def issue(t, b):
        pltpu.async_copy(q_hbm.at[dstall.at[pl.ds(t * CHA, CHA)]],
                         qbufs[b], qsems[b])
        pltpu.async_copy(k_hbm.at[srcall.at[pl.ds(t * CHA, CHA)]],
                         kbufs[b], ksems[b])

    def wait(t, b):
        pltpu.make_async_copy(q_hbm.at[dstall.at[pl.ds(t * CHA, CHA)]],
                              qbufs[b], qsems[b]).wait()
        pltpu.make_async_copy(k_hbm.at[srcall.at[pl.ds(t * CHA, CHA)]],
                              kbufs[b], ksems[b]).wait()

    issue(0, 0)
    issue(1, 1)
    issue(2, 2)

    def quad_body(tt, rmax):
        for b in range(4):
            t = tt * 4 + b
            issue(jnp.minimum(t + 3, NCHA - 1), (b + 3) % 4)
            wait(t, b)
            qr = qbufs[b]
            kr = kbufs[b]

            def group_body(g, rmax):
                accs = []
                for j in range(16):
                    r = g * 16 + j
                    a0 = qr[r, pl.ds(0, 16)] * kr[r, pl.ds(0, 16)]
                    a1 = qr[r, pl.ds(16, 16)] * kr[r, pl.ds(16, 16)]
                    for blk in range(2, D // 16, 2):
                        a0 = a0 + qr[r, pl.ds(blk * 16, 16)] * \
                            kr[r, pl.ds(blk * 16, 16)]
                        a1 = a1 + qr[r, pl.ds(blk * 16 + 16, 16)] * \
                            kr[r, pl.ds(blk * 16 + 16, 16)]
                    accs.append(a0 + a1)
                dvs = [jnp.sum(a) for a in accs]
                parts = [jnp.where(lanes == j, dvs[j], 0.0)
                         for j in range(16)]
                while len(parts) > 1:
                    parts = [parts[i] + parts[i + 1]
                             for i in range(0, len(parts), 2)]
                lg = parts[0] * inv
                lall[pl.ds(t * CHA + g * 16, 16)] = lg
                return jnp.maximum(rmax, lg)

            rmax = lax.fori_loop(0, CHA // 16, group_body, rmax)
        return rmax

    rmax = lax.fori_loop(0, NCHA // 4, quad_body,
                         jnp.full((16,), -1e30, jnp.float32))
    wait(NCHA - 1, 0)
    wait(NCHA - 1, 1)
    wait(NCHA - 1, 2)
    pltpu.sync_copy(lall, logits_hbm.at[pl.ds(base, EWP)])
    mxbuf[...] = rmax
    pltpu.sync_copy(mxbuf, maxes_hbm.at[wid])
    pltpu.sync_copy(mxbuf, mx_sh.at[sid])

    plsc.subcore_barrier()

    pltpu.sync_copy(mx_sh, mxv)
    mc = mxv[0, pl.ds(0, 16)]
    for i in range(1, NS):
        mc = jnp.maximum(mc, mxv[i, pl.ds(0, 16)])
    mcs = jnp.max(mc)

    def chunk_body(t, carry):
        for g in range(CHA // 16):
            sl = pl.ds(g * 16, 16)
            dsti2[sl] = dstall[pl.ds(t * CHA + g * 16, 16)]
            echunk[sl] = jnp.exp(lall[pl.ds(t * CHA + g * 16, 16)] - mcs)
        pltpu.sync_copy(echunk, s_acc.at[dsti2], add=True)
        return carry

    lax.fori_loop(0, NCHA, chunk_body, 0)
    plsc.subcore_barrier()

    @pl.when(sid == 0)
    def _dump():
        pltpu.sync_copy(s_acc, zvm)
        pltpu.sync_copy(zvm, sums_hbm.at[cid])


@functools.partial(
    pl.kernel,
    out_type=jax.ShapeDtypeStruct((EP,), jnp.float32),
    mesh=_mesh,
    compiler_params=_params,
    scratch_types=[
        pltpu.VMEM((EWP,), jnp.int32),
        pltpu.VMEM((EWP,), jnp.float32),
        pltpu.VMEM((NP,), jnp.float32),
        pltpu.VMEM((NW * 16,), jnp.float32),
        pltpu.VMEM((CH,), jnp.float32),
    ],
)
def _sc_alpha(logits_hbm, dst_hbm, maxes_hbm, sums_hbm, alpha_hbm,
              dstall, lall, s_full, maxes_vm, sbuf):
    wid = _worker_id()
    base = wid * EWP

    pltpu.sync_copy(maxes_hbm, maxes_vm)
    m0v = maxes_vm[pl.ds(0, 16)]
    m1v = maxes_vm[pl.ds(NS * 16, 16)]
    for i in range(1, NS):
        m0v = jnp.maximum(m0v, maxes_vm[pl.ds(i * 16, 16)])
        m1v = jnp.maximum(m1v, maxes_vm[pl.ds((NS + i) * 16, 16)])
    m0 = jnp.max(m0v)
    m1 = jnp.max(m1v)
    gmax = jnp.maximum(m0, m1)
    e0 = jnp.exp(jnp.full((16,), m0 - gmax, jnp.float32))
    e1 = jnp.exp(jnp.full((16,), m1 - gmax, jnp.float32))
    pltpu.sync_copy(dst_hbm.at[pl.ds(base, EWP)], dstall)
    pltpu.sync_copy(logits_hbm.at[pl.ds(base, EWP)], lall)

    pltpu.sync_copy(sums_hbm.at[0], s_full)
    eps = jnp.float32(1e-16)

    def comb(i, carry):
        pltpu.sync_copy(sums_hbm.at[1, pl.ds(i * CH, CH)], sbuf)
        for g in range(CH // 16):
            sl = pl.ds(i * CH + g * 16, 16)
            s_full[sl] = s_full[sl] * e0 + sbuf[pl.ds(g * 16, 16)] * e1 + eps
        return carry

    lax.fori_loop(0, NP // CH, comb, 0)

    def seg_body(i, carry):
        sl = pl.ds(i * 16, 16)
        dst16 = dstall[sl]
        sv = plsc.load_gather(s_full, [dst16])
        lall[sl] = jnp.exp(lall[sl] - gmax) / sv
        return carry

    lax.fori_loop(0, EWP // 16, seg_body, 0)
    pltpu.sync_copy(lall, alpha_hbm.at[pl.ds(base, EWP)])


CHC = 80
NCHC = EWP // CHC


@functools.partial(
    pl.kernel,
    out_type=jax.ShapeDtypeStruct((NC, NP, D), jnp.float32),
    mesh=_mesh,
    compiler_params=_params,
    scratch_types=(
        [pltpu.VMEM_SHARED((NP, D), jnp.float32)] +
        [pltpu.VMEM((CHC,), jnp.int32),
         pltpu.VMEM((CHC,), jnp.int32),
         pltpu.VMEM((CHC,), jnp.float32)] * 2 +
        [pltpu.VMEM((CHC, D), jnp.float32)] * 2 +
        [pltpu.VMEM((CHC, D), jnp.float32)] * 2 +
        [pltpu.VMEM((CHC,), jnp.int32)] * 2 +
        [pltpu.SemaphoreType.DMA] * 6
    ),
)
def _sc_aggregate(v_hbm, src_hbm, dst_hbm, alpha_hbm, parts_hbm,
                  out_acc,
                  srci0, dsti0, alf0, srci1, dsti1, alf1,
                  vrows0, vrows1, srows0, srows1, sdst0, sdst1,
                  ms0, ms1, gs0, gs1, ss0, ss1):
    cid = lax.axis_index("c")
    sid = lax.axis_index("s")
    wid = cid * NS + sid
    base = wid * EWP

    srcis = (srci0, srci1)
    dstis = (dsti0, dsti1)
    alfs = (alf0, alf1)
    vbufs = (vrows0, vrows1)
    srows = (srows0, srows1)
    sdsts = (sdst0, sdst1)
    msems = (ms0, ms1)
    gsems = (gs0, gs1)
    ssems = (ss0, ss1)

    def meta_issue(t, b):
        off = base + t * CHC
        pltpu.async_copy(src_hbm.at[pl.ds(off, CHC)], srcis[b], msems[b])
        pltpu.async_copy(dst_hbm.at[pl.ds(off, CHC)], dstis[b], msems[b])
        pltpu.async_copy(alpha_hbm.at[pl.ds(off, CHC)], alfs[b], msems[b])

    def meta_wait(t, b):
        off = base + t * CHC
        pltpu.make_async_copy(src_hbm.at[pl.ds(off, CHC)],
                              srcis[b], msems[b]).wait()
        pltpu.make_async_copy(dst_hbm.at[pl.ds(off, CHC)],
                              dstis[b], msems[b]).wait()
        pltpu.make_async_copy(alpha_hbm.at[pl.ds(off, CHC)],
                              alfs[b], msems[b]).wait()

    def gather_issue(b):
        pltpu.async_copy(v_hbm.at[srcis[b]], vbufs[b], gsems[b])

    def gather_wait(b):
        pltpu.make_async_copy(v_hbm.at[srcis[b]], vbufs[b], gsems[b]).wait()

    def scatter_issue(b):
        pltpu.async_copy(srows[b], out_acc.at[sdsts[b]], ssems[b], add=True)

    def scatter_wait(b):
        pltpu.make_async_copy(srows[b], out_acc.at[sdsts[b]],
                              ssems[b]).wait()

    z16 = jnp.zeros((16,), jnp.float32)
    zi16 = jnp.zeros((16,), jnp.int32)

    def zb(i, carry):
        for blk in range(D // 16):
            vrows0[i, pl.ds(blk * 16, 16)] = z16
        return carry

    lax.fori_loop(0, CHC, zb, 0)

    def zcopy(i, carry):
        pltpu.sync_copy(vrows0.at[pl.ds(0, ROWS_W // 8)],
                        out_acc.at[pl.ds(sid * ROWS_W + i * (ROWS_W // 8),
                                         ROWS_W // 8)])
        return carry

    lax.fori_loop(0, 8, zcopy, 0)

    def zsc(i, carry):
        for blk in range(D // 16):
            srows0[i, pl.ds(blk * 16, 16)] = z16
            srows1[i, pl.ds(blk * 16, 16)] = z16
        return carry

    lax.fori_loop(0, CHC, zsc, 0)
    for g in range(CHC // 16):
        sdst0[pl.ds(g * 16, 16)] = zi16
        sdst1[pl.ds(g * 16, 16)] = zi16

    plsc.subcore_barrier()

    scatter_issue(0)
    scatter_issue(1)

    meta_issue(0, 0)
    meta_wait(0, 0)
    gather_issue(0)
    meta_issue(1, 1)

    def pair_body(tt, carry):
        for b in range(2):
            t = tt * 2 + b
            ob = 1 - b
            gather_wait(b)
            meta_wait(jnp.minimum(t + 1, NCHC - 1), ob)
            gather_issue(ob)
            scatter_wait(b)

            def group_scale(g, carry):
                a16 = alfs[b][pl.ds(g * 16, 16)]
                sdsts[b][pl.ds(g * 16, 16)] = dstis[b][pl.ds(g * 16, 16)]
                for j in range(16):
                    a = a16[j]
                    for blk in range(D // 16):
                        sl = pl.ds(blk * 16, 16)
                        srows[b][g * 16 + j, sl] =                             vbufs[b][g * 16 + j, sl] * a
                return carry

            lax.fori_loop(0, CHC // 16, group_scale, 0)
            scatter_issue(b)
            meta_issue(jnp.minimum(t + 2, NCHC - 1), b)
        return carry

    lax.fori_loop(0, NCHC // 2, pair_body, 0)
    gather_wait(0)
    meta_wait(NCHC - 1, 1)
    scatter_wait(0)
    scatter_wait(1)
    plsc.subcore_barrier()

    pltpu.sync_copy(out_acc.at[pl.ds(sid * ROWS_W, ROWS_W)],
                    parts_hbm.at[cid, pl.ds(sid * ROWS_W, ROWS_W)])


_BLK = 2528


def _proj1_body(x_ref, w_ref, b_ref, q_ref, k_ref, v_ref, s_ref):
    r = jnp.dot(x_ref[...], w_ref[...],
                preferred_element_type=jnp.float32) + b_ref[...]
    q_ref[...] = r[:, 0 * D:1 * D]
    k_ref[...] = r[:, 1 * D:2 * D]
    v_ref[...] = r[:, 2 * D:3 * D]
    s_ref[...] = r[:, 3 * D:4 * D]


def _proj2_body(p0_ref, p1_ref, sk_ref, w_ref, b_ref,
                q_ref, k_ref, v_ref, s_ref):
    h = jnp.maximum(p0_ref[0] + p1_ref[0] + sk_ref[...], 0.0)
    r = jnp.dot(h, w_ref[...], preferred_element_type=jnp.float32) + b_ref[...]
    q_ref[...] = r[:, 0 * D:1 * D]
    k_ref[...] = r[:, 1 * D:2 * D]
    v_ref[...] = r[:, 2 * D:3 * D]
    s_ref[...] = r[:, 3 * D:4 * D]


def _combine_body(p0_ref, p1_ref, sk_ref, o_ref):
    o_ref[...] = p0_ref[0] + p1_ref[0] + sk_ref[...]


def _pad_edges_body(e_ref, s_ref, d_ref):
    s_ref[pl.ds(0, E // 128)] = e_ref[0]
    s_ref[pl.ds(E // 128, (EP - E) // 128)] = jnp.zeros(
        ((EP - E) // 128, 128), jnp.int32)
    d_ref[pl.ds(0, E // 128)] = e_ref[1]
    d_ref[pl.ds(E // 128, (EP - E) // 128)] = jnp.full(
        ((EP - E) // 128, 128), N, jnp.int32)


def _pad_x_body(x_ref, o_ref):
    o_ref[pl.ds(0, N)] = x_ref[...]
    o_ref[pl.ds(N, NP - N)] = jnp.zeros((NP - N, D), jnp.float32)


def _pad_edges(edge_index):
    s2d, d2d = pl.pallas_call(
        _pad_edges_body,
        in_specs=[pl.BlockSpec((2, E // 128, 128), lambda: (0, 0, 0))],
        out_specs=[pl.BlockSpec((EP // 128, 128), lambda: (0, 0))] * 2,
        out_shape=[jax.ShapeDtypeStruct((EP // 128, 128), jnp.int32)] * 2,
    )(edge_index.reshape(2, E // 128, 128))
    return s2d.reshape(EP), d2d.reshape(EP)


def _pad_x(x):
    return pl.pallas_call(
        _pad_x_body,
        in_specs=[pl.BlockSpec((N, D), lambda: (0, 0))],
        out_specs=pl.BlockSpec((NP, D), lambda: (0, 0)),
        out_shape=jax.ShapeDtypeStruct((NP, D), jnp.float32),
    )(x)


def _proj1(x, wcat, bcat):
    return pl.pallas_call(
        _proj1_body,
        grid=(NP // _BLK,),
        in_specs=[
            pl.BlockSpec((_BLK, D), lambda i: (i, 0)),
            pl.BlockSpec((D, 4 * D), lambda i: (0, 0)),
            pl.BlockSpec((1, 4 * D), lambda i: (0, 0)),
        ],
        out_specs=[pl.BlockSpec((_BLK, D), lambda i: (i, 0))] * 4,
        out_shape=[jax.ShapeDtypeStruct((NP, D), jnp.float32)] * 4,
    )(x, wcat, bcat)


def _proj2(parts, sk, wcat, bcat):
    return pl.pallas_call(
        _proj2_body,
        grid=(NP // _BLK,),
        in_specs=[
            pl.BlockSpec((1, _BLK, D), lambda i: (0, i, 0)),
            pl.BlockSpec((1, _BLK, D), lambda i: (1, i, 0)),
            pl.BlockSpec((_BLK, D), lambda i: (i, 0)),
            pl.BlockSpec((D, 4 * D), lambda i: (0, 0)),
            pl.BlockSpec((1, 4 * D), lambda i: (0, 0)),
        ],
        out_specs=[pl.BlockSpec((_BLK, D), lambda i: (i, 0))] * 4,
        out_shape=[jax.ShapeDtypeStruct((NP, D), jnp.float32)] * 4,
    )(parts, parts, sk, wcat, bcat)


def _combine(parts, sk):
    return pl.pallas_call(
        _combine_body,
        grid=(5,),
        in_specs=[
            pl.BlockSpec((1, N // 5, D), lambda i: (0, i, 0)),
            pl.BlockSpec((1, N // 5, D), lambda i: (1, i, 0)),
            pl.BlockSpec((N // 5, D), lambda i: (i, 0)),
        ],
        out_specs=pl.BlockSpec((N // 5, D), lambda i: (i, 0)),
        out_shape=jax.ShapeDtypeStruct((N, D), jnp.float32),
    )(parts, parts, sk)


def _edge_softmax_aggregate(q, k, v, srcp, dstp):
    logits, maxes, sums = _sc_logsums(q, k, srcp, dstp)
    alpha = _sc_alpha(logits, dstp, maxes.reshape(-1), sums)
    return _sc_aggregate(v, srcp, dstp, alpha)


def kernel(x, edge_index, Wq1, bq1, Wk1, bk1, Wv1, bv1, Ws1, bs1,
           Wq2, bq2, Wk2, bk2, Wv2, bv2, Ws2, bs2):
    srcp, dstp = _pad_edges(edge_index)
    xp = _pad_x(x)

    w1 = jnp.concatenate([Wq1, Wk1, Wv1, Ws1], axis=1)
    b1 = jnp.concatenate([bq1, bk1, bv1, bs1]).reshape(1, -1)
    w2 = jnp.concatenate([Wq2, Wk2, Wv2, Ws2], axis=1)
    b2 = jnp.concatenate([bq2, bk2, bv2, bs2]).reshape(1, -1)

    q1, k1, v1, sk1 = _proj1(xp, w1, b1)
    parts = _edge_softmax_aggregate(q1, k1, v1, srcp, dstp)
    q2, k2, v2, sk2 = _proj2(parts, sk1, w2, b2)
    parts = _edge_softmax_aggregate(q2, k2, v2, srcp, dstp)
    return _combine(parts, sk2)


# speedup vs baseline: 1.0930x; 1.0100x over previous
import functools

import jax
import jax.numpy as jnp
import numpy as np
from jax import lax
from jax.experimental import pallas as pl
from jax.experimental.pallas import tpu as pltpu
from jax.experimental.pallas import tpu_sc as plsc

N = 10000
E = 320000
D = 128
H = 128

NC = 2
NS = 16
NW = NC * NS
NP = 10112
EWP = 10240
EP = EWP * NW
CH = 128
NCH = EWP // CH
ROWS_W = NP // NS
INV_SQRT_H = float(1.0 / np.sqrt(H))

_mesh = plsc.VectorSubcoreMesh(core_axis_name="c", subcore_axis_name="s")
_params = pltpu.CompilerParams(needs_layout_passes=False)


def _worker_id():
    return lax.axis_index("c") * NS + lax.axis_index("s")


def _global_max(maxes_vm):
    rmax = maxes_vm[pl.ds(0, 16)]
    for i in range(1, NW):
        rmax = jnp.maximum(rmax, maxes_vm[pl.ds(i * 16, 16)])
    return jnp.max(rmax)


CHA = 64
NCHA = EWP // CHA


@functools.partial(
    pl.kernel,
    out_type=[
        jax.ShapeDtypeStruct((EP,), jnp.float32),
        jax.ShapeDtypeStruct((NW, 16), jnp.float32),
        jax.ShapeDtypeStruct((NC, NP), jnp.float32),
    ],
    mesh=_mesh,
    compiler_params=_params,
    scratch_types=(
        [pltpu.VMEM((EWP,), jnp.int32)] * 2 +
        [pltpu.VMEM((EWP,), jnp.float32)] +
        [pltpu.VMEM((CHA, D), jnp.float32)] * 8 +
        [pltpu.VMEM((16,), jnp.float32)] +
        [pltpu.VMEM_SHARED((NP,), jnp.float32)] +
        [pltpu.VMEM_SHARED((NS, 16), jnp.float32)] +
        [pltpu.VMEM((NP,), jnp.float32)] +
        [pltpu.VMEM((NS, 16), jnp.float32)] +
        [pltpu.VMEM((CHA,), jnp.int32)] * 2 +
        [pltpu.VMEM((CHA,), jnp.float32)] * 2 +
        [pltpu.SemaphoreType.DMA] * 10
    ),
)
def _sc_logsums(q_hbm, k_hbm, src_hbm, dst_hbm,
                logits_hbm, maxes_hbm, sums_hbm,
                srcall, dstall, lall, q0, k0, q1, k1, q2, k2, q3, k3, mxbuf,
                s_acc, mx_sh, zvm, mxv, dsti2a, dsti2b, echunka, echunkb,
                qs0, ks0, qs1, ks1, qs2, ks2, qs3, ks3, es0, es1):
    cid = lax.axis_index("c")
    sid = lax.axis_index("s")
    wid = cid * NS + sid
    base = wid * EWP
    lanes = lax.iota(jnp.int32, 16)
    inv = jnp.float32(INV_SQRT_H)

    @pl.when(sid == 0)
    def _zero():
        z16 = jnp.zeros((16,), jnp.float32)

        def zb(i, carry):
            zvm[pl.ds(i * 16, 16)] = z16
            return carry

        lax.fori_loop(0, NP // 16, zb, 0)
        pltpu.sync_copy(zvm, s_acc)

    pltpu.sync_copy(src_hbm.at[pl.ds(base, EWP)], srcall)
    pltpu.sync_copy(dst_hbm.at[pl.ds(base, EWP)], dstall)

    qbufs = (q0, q1, q2, q3)
    kbufs = (k0, k1, k2, k3)
    qsems = (qs0, qs1, qs2, qs3)
    ksems = (ks0, ks1, ks2, ks3)

    def issue(t, b):
        pltpu.async_copy(q_hbm.at[dstall.at[pl.ds(t * CHA, CHA)]],
                         qbufs[b], qsems[b])
        pltpu.async_copy(k_hbm.at[srcall.at[pl.ds(t * CHA, CHA)]],
                         kbufs[b], ksems[b])

    def wait(t, b):
        pltpu.make_async_copy(q_hbm.at[dstall.at[pl.ds(t * CHA, CHA)]],
                              qbufs[b], qsems[b]).wait()
        pltpu.make_async_copy(k_hbm.at[srcall.at[pl.ds(t * CHA, CHA)]],
                              kbufs[b], ksems[b]).wait()

    issue(0, 0)
    issue(1, 1)
    issue(2, 2)

    def quad_body(tt, rmax):
        for b in range(4):
            t = tt * 4 + b
            issue(jnp.minimum(t + 3, NCHA - 1), (b + 3) % 4)
            wait(t, b)
            qr = qbufs[b]
            kr = kbufs[b]

            def group_body(g, rmax):
                accs = []
                for j in range(16):
                    r = g * 16 + j
                    a0 = qr[r, pl.ds(0, 16)] * kr[r, pl.ds(0, 16)]
                    a1 = qr[r, pl.ds(16, 16)] * kr[r, pl.ds(16, 16)]
                    for blk in range(2, D // 16, 2):
                        a0 = a0 + qr[r, pl.ds(blk * 16, 16)] * \
                            kr[r, pl.ds(blk * 16, 16)]
                        a1 = a1 + qr[r, pl.ds(blk * 16 + 16, 16)] * \
                            kr[r, pl.ds(blk * 16 + 16, 16)]
                    accs.append(a0 + a1)
                dvs = [jnp.sum(a) for a in accs]
                parts = [jnp.where(lanes == j, dvs[j], 0.0)
                         for j in range(16)]
                while len(parts) > 1:
                    parts = [parts[i] + parts[i + 1]
                             for i in range(0, len(parts), 2)]
                lg = parts[0] * inv
                lall[pl.ds(t * CHA + g * 16, 16)] = lg
                return jnp.maximum(rmax, lg)

            rmax = lax.fori_loop(0, CHA // 16, group_body, rmax)
        return rmax

    rmax = lax.fori_loop(0, NCHA // 4, quad_body,
                         jnp.full((16,), -1e30, jnp.float32))
    wait(NCHA - 1, 0)
    wait(NCHA - 1, 1)
    wait(NCHA - 1, 2)
    pltpu.sync_copy(lall, logits_hbm.at[pl.ds(base, EWP)])
    mxbuf[...] = rmax
    pltpu.sync_copy(mxbuf, maxes_hbm.at[wid])
    pltpu.sync_copy(mxbuf, mx_sh.at[sid])

    plsc.subcore_barrier()

    pltpu.sync_copy(mx_sh, mxv)
    mc = mxv[0, pl.ds(0, 16)]
    for i in range(1, NS):
        mc = jnp.maximum(mc, mxv[i, pl.ds(0, 16)])
    mcs = jnp.max(mc)

    dbufs = (dsti2a, dsti2b)
    ebufs = (echunka, echunkb)
    esems = (es0, es1)

    def sc_issue(b):
        pltpu.async_copy(ebufs[b], s_acc.at[dbufs[b]], esems[b], add=True)

    def sc_wait(b):
        pltpu.make_async_copy(ebufs[b], s_acc.at[dbufs[b]], esems[b]).wait()

    def fill(t, b):
        for g in range(CHA // 16):
            sl = pl.ds(g * 16, 16)
            dbufs[b][sl] = dstall[pl.ds(t * CHA + g * 16, 16)]
            ebufs[b][sl] = jnp.exp(lall[pl.ds(t * CHA + g * 16, 16)] - mcs)

    fill(0, 0)
    sc_issue(0)
    fill(1, 1)
    sc_issue(1)

    def spair_body(tt, carry):
        for b in range(2):
            t = tt * 2 + b
            sc_wait(b)
            fill(jnp.minimum(t + 2, NCHA - 1), b)
            sc_issue(b)
        return carry

    lax.fori_loop(0, (NCHA - 2) // 2, spair_body, 0)
    sc_wait(0)
    sc_wait(1)
    plsc.subcore_barrier()

    @pl.when(sid == 0)
    def _dump():
        pltpu.sync_copy(s_acc, zvm)
        pltpu.sync_copy(zvm, sums_hbm.at[cid])


@functools.partial(
    pl.kernel,
    out_type=jax.ShapeDtypeStruct((EP,), jnp.float32),
    mesh=_mesh,
    compiler_params=_params,
    scratch_types=[
        pltpu.VMEM((EWP,), jnp.int32),
        pltpu.VMEM((EWP,), jnp.float32),
        pltpu.VMEM((NP,), jnp.float32),
        pltpu.VMEM((NW * 16,), jnp.float32),
        pltpu.VMEM((CH,), jnp.float32),
    ],
)
def _sc_alpha(logits_hbm, dst_hbm, maxes_hbm, sums_hbm, alpha_hbm,
              dstall, lall, s_full, maxes_vm, sbuf):
    wid = _worker_id()
    base = wid * EWP

    pltpu.sync_copy(maxes_hbm, maxes_vm)
    m0v = maxes_vm[pl.ds(0, 16)]
    m1v = maxes_vm[pl.ds(NS * 16, 16)]
    for i in range(1, NS):
        m0v = jnp.maximum(m0v, maxes_vm[pl.ds(i * 16, 16)])
        m1v = jnp.maximum(m1v, maxes_vm[pl.ds((NS + i) * 16, 16)])
    m0 = jnp.max(m0v)
    m1 = jnp.max(m1v)
    gmax = jnp.maximum(m0, m1)
    e0 = jnp.exp(jnp.full((16,), m0 - gmax, jnp.float32))
    e1 = jnp.exp(jnp.full((16,), m1 - gmax, jnp.float32))
    pltpu.sync_copy(dst_hbm.at[pl.ds(base, EWP)], dstall)
    pltpu.sync_copy(logits_hbm.at[pl.ds(base, EWP)], lall)

    pltpu.sync_copy(sums_hbm.at[0], s_full)
    eps = jnp.float32(1e-16)

    def comb(i, carry):
        pltpu.sync_copy(sums_hbm.at[1, pl.ds(i * CH, CH)], sbuf)
        for g in range(CH // 16):
            sl = pl.ds(i * CH + g * 16, 16)
            s_full[sl] = s_full[sl] * e0 + sbuf[pl.ds(g * 16, 16)] * e1 + eps
        return carry

    lax.fori_loop(0, NP // CH, comb, 0)

    def seg_body(i, carry):
        sl = pl.ds(i * 16, 16)
        dst16 = dstall[sl]
        sv = plsc.load_gather(s_full, [dst16])
        lall[sl] = jnp.exp(lall[sl] - gmax) / sv
        return carry

    lax.fori_loop(0, EWP // 16, seg_body, 0)
    pltpu.sync_copy(lall, alpha_hbm.at[pl.ds(base, EWP)])


CHC = 80
NCHC = EWP // CHC


@functools.partial(
    pl.kernel,
    out_type=jax.ShapeDtypeStruct((NC, NP, D), jnp.float32),
    mesh=_mesh,
    compiler_params=_params,
    scratch_types=(
        [pltpu.VMEM_SHARED((NP, D), jnp.float32)] +
        [pltpu.VMEM((CHC,), jnp.int32),
         pltpu.VMEM((CHC,), jnp.int32),
         pltpu.VMEM((CHC,), jnp.float32)] * 2 +
        [pltpu.VMEM((CHC, D), jnp.float32)] * 2 +
        [pltpu.VMEM((CHC, D), jnp.float32)] * 2 +
        [pltpu.VMEM((CHC,), jnp.int32)] * 2 +
        [pltpu.SemaphoreType.DMA] * 6
    ),
)
def _sc_aggregate(v_hbm, src_hbm, dst_hbm, alpha_hbm, parts_hbm,
                  out_acc,
                  srci0, dsti0, alf0, srci1, dsti1, alf1,
                  vrows0, vrows1, srows0, srows1, sdst0, sdst1,
                  ms0, ms1, gs0, gs1, ss0, ss1):
    cid = lax.axis_index("c")
    sid = lax.axis_index("s")
    wid = cid * NS + sid
    base = wid * EWP

    srcis = (srci0, srci1)
    dstis = (dsti0, dsti1)
    alfs = (alf0, alf1)
    vbufs = (vrows0, vrows1)
    srows = (srows0, srows1)
    sdsts = (sdst0, sdst1)
    msems = (ms0, ms1)
    gsems = (gs0, gs1)
    ssems = (ss0, ss1)

    def meta_issue(t, b):
        off = base + t * CHC
        pltpu.async_copy(src_hbm.at[pl.ds(off, CHC)], srcis[b], msems[b])
        pltpu.async_copy(dst_hbm.at[pl.ds(off, CHC)], dstis[b], msems[b])
        pltpu.async_copy(alpha_hbm.at[pl.ds(off, CHC)], alfs[b], msems[b])

    def meta_wait(t, b):
        off = base + t * CHC
        pltpu.make_async_copy(src_hbm.at[pl.ds(off, CHC)],
                              srcis[b], msems[b]).wait()
        pltpu.make_async_copy(dst_hbm.at[pl.ds(off, CHC)],
                              dstis[b], msems[b]).wait()
        pltpu.make_async_copy(alpha_hbm.at[pl.ds(off, CHC)],
                              alfs[b], msems[b]).wait()

    def gather_issue(b):
        pltpu.async_copy(v_hbm.at[srcis[b]], vbufs[b], gsems[b])

    def gather_wait(b):
        pltpu.make_async_copy(v_hbm.at[srcis[b]], vbufs[b], gsems[b]).wait()

    def scatter_issue(b):
        pltpu.async_copy(srows[b], out_acc.at[sdsts[b]], ssems[b], add=True)

    def scatter_wait(b):
        pltpu.make_async_copy(srows[b], out_acc.at[sdsts[b]],
                              ssems[b]).wait()

    z16 = jnp.zeros((16,), jnp.float32)
    zi16 = jnp.zeros((16,), jnp.int32)

    def zb(i, carry):
        for blk in range(D // 16):
            vrows0[i, pl.ds(blk * 16, 16)] = z16
        return carry

    lax.fori_loop(0, CHC, zb, 0)

    def zcopy(i, carry):
        pltpu.sync_copy(vrows0.at[pl.ds(0, ROWS_W // 8)],
                        out_acc.at[pl.ds(sid * ROWS_W + i * (ROWS_W // 8),
                                         ROWS_W // 8)])
        return carry

    lax.fori_loop(0, 8, zcopy, 0)

    def zsc(i, carry):
        for blk in range(D // 16):
            srows0[i, pl.ds(blk * 16, 16)] = z16
            srows1[i, pl.ds(blk * 16, 16)] = z16
        return carry

    lax.fori_loop(0, CHC, zsc, 0)
    for g in range(CHC // 16):
        sdst0[pl.ds(g * 16, 16)] = zi16
        sdst1[pl.ds(g * 16, 16)] = zi16

    plsc.subcore_barrier()

    scatter_issue(0)
    scatter_issue(1)

    meta_issue(0, 0)
    meta_wait(0, 0)
    gather_issue(0)
    meta_issue(1, 1)

    def pair_body(tt, carry):
        for b in range(2):
            t = tt * 2 + b
            ob = 1 - b
            gather_wait(b)
            meta_wait(jnp.minimum(t + 1, NCHC - 1), ob)
            gather_issue(ob)
            scatter_wait(b)

            def group_scale(g, carry):
                a16 = alfs[b][pl.ds(g * 16, 16)]
                sdsts[b][pl.ds(g * 16, 16)] = dstis[b][pl.ds(g * 16, 16)]
                for j in range(16):
                    a = a16[j]
                    for blk in range(D // 16):
                        sl = pl.ds(blk * 16, 16)
                        srows[b][g * 16 + j, sl] =                             vbufs[b][g * 16 + j, sl] * a
                return carry

            lax.fori_loop(0, CHC // 16, group_scale, 0)
            scatter_issue(b)
            meta_issue(jnp.minimum(t + 2, NCHC - 1), b)
        return carry

    lax.fori_loop(0, NCHC // 2, pair_body, 0)
    gather_wait(0)
    meta_wait(NCHC - 1, 1)
    scatter_wait(0)
    scatter_wait(1)
    plsc.subcore_barrier()

    pltpu.sync_copy(out_acc.at[pl.ds(sid * ROWS_W, ROWS_W)],
                    parts_hbm.at[cid, pl.ds(sid * ROWS_W, ROWS_W)])


_BLK = 2528


def _proj1_body(x_ref, w_ref, b_ref, q_ref, k_ref, v_ref, s_ref):
    r = jnp.dot(x_ref[...], w_ref[...],
                preferred_element_type=jnp.float32) + b_ref[...]
    q_ref[...] = r[:, 0 * D:1 * D]
    k_ref[...] = r[:, 1 * D:2 * D]
    v_ref[...] = r[:, 2 * D:3 * D]
    s_ref[...] = r[:, 3 * D:4 * D]


def _proj2_body(p0_ref, p1_ref, sk_ref, w_ref, b_ref,
                q_ref, k_ref, v_ref, s_ref):
    h = jnp.maximum(p0_ref[0] + p1_ref[0] + sk_ref[...], 0.0)
    r = jnp.dot(h, w_ref[...], preferred_element_type=jnp.float32) + b_ref[...]
    q_ref[...] = r[:, 0 * D:1 * D]
    k_ref[...] = r[:, 1 * D:2 * D]
    v_ref[...] = r[:, 2 * D:3 * D]
    s_ref[...] = r[:, 3 * D:4 * D]


def _combine_body(p0_ref, p1_ref, sk_ref, o_ref):
    o_ref[...] = p0_ref[0] + p1_ref[0] + sk_ref[...]


def _pad_edges_body(e_ref, s_ref, d_ref):
    s_ref[pl.ds(0, E // 128)] = e_ref[0]
    s_ref[pl.ds(E // 128, (EP - E) // 128)] = jnp.zeros(
        ((EP - E) // 128, 128), jnp.int32)
    d_ref[pl.ds(0, E // 128)] = e_ref[1]
    d_ref[pl.ds(E // 128, (EP - E) // 128)] = jnp.full(
        ((EP - E) // 128, 128), N, jnp.int32)


def _pad_x_body(x_ref, o_ref):
    o_ref[pl.ds(0, N)] = x_ref[...]
    o_ref[pl.ds(N, NP - N)] = jnp.zeros((NP - N, D), jnp.float32)


def _pad_edges(edge_index):
    s2d, d2d = pl.pallas_call(
        _pad_edges_body,
        in_specs=[pl.BlockSpec((2, E // 128, 128), lambda: (0, 0, 0))],
        out_specs=[pl.BlockSpec((EP // 128, 128), lambda: (0, 0))] * 2,
        out_shape=[jax.ShapeDtypeStruct((EP // 128, 128), jnp.int32)] * 2,
    )(edge_index.reshape(2, E // 128, 128))
    return s2d.reshape(EP), d2d.reshape(EP)


def _pad_x(x):
    return pl.pallas_call(
        _pad_x_body,
        in_specs=[pl.BlockSpec((N, D), lambda: (0, 0))],
        out_specs=pl.BlockSpec((NP, D), lambda: (0, 0)),
        out_shape=jax.ShapeDtypeStruct((NP, D), jnp.float32),
    )(x)


def _proj1(x, wcat, bcat):
    return pl.pallas_call(
        _proj1_body,
        grid=(NP // _BLK,),
        in_specs=[
            pl.BlockSpec((_BLK, D), lambda i: (i, 0)),
            pl.BlockSpec((D, 4 * D), lambda i: (0, 0)),
            pl.BlockSpec((1, 4 * D), lambda i: (0, 0)),
        ],
        out_specs=[pl.BlockSpec((_BLK, D), lambda i: (i, 0))] * 4,
        out_shape=[jax.ShapeDtypeStruct((NP, D), jnp.float32)] * 4,
    )(x, wcat, bcat)


def _proj2(parts, sk, wcat, bcat):
    return pl.pallas_call(
        _proj2_body,
        grid=(NP // _BLK,),
        in_specs=[
            pl.BlockSpec((1, _BLK, D), lambda i: (0, i, 0)),
            pl.BlockSpec((1, _BLK, D), lambda i: (1, i, 0)),
            pl.BlockSpec((_BLK, D), lambda i: (i, 0)),
            pl.BlockSpec((D, 4 * D), lambda i: (0, 0)),
            pl.BlockSpec((1, 4 * D), lambda i: (0, 0)),
        ],
        out_specs=[pl.BlockSpec((_BLK, D), lambda i: (i, 0))] * 4,
        out_shape=[jax.ShapeDtypeStruct((NP, D), jnp.float32)] * 4,
    )(parts, parts, sk, wcat, bcat)


def _combine(parts, sk):
    return pl.pallas_call(
        _combine_body,
        grid=(5,),
        in_specs=[
            pl.BlockSpec((1, N // 5, D), lambda i: (0, i, 0)),
            pl.BlockSpec((1, N // 5, D), lambda i: (1, i, 0)),
            pl.BlockSpec((N // 5, D), lambda i: (i, 0)),
        ],
        out_specs=pl.BlockSpec((N // 5, D), lambda i: (i, 0)),
        out_shape=jax.ShapeDtypeStruct((N, D), jnp.float32),
    )(parts, parts, sk)


def _edge_softmax_aggregate(q, k, v, srcp, dstp):
    logits, maxes, sums = _sc_logsums(q, k, srcp, dstp)
    alpha = _sc_alpha(logits, dstp, maxes.reshape(-1), sums)
    return _sc_aggregate(v, srcp, dstp, alpha)


def kernel(x, edge_index, Wq1, bq1, Wk1, bk1, Wv1, bv1, Ws1, bs1,
           Wq2, bq2, Wk2, bk2, Wv2, bv2, Ws2, bs2):
    srcp, dstp = _pad_edges(edge_index)
    xp = _pad_x(x)

    w1 = jnp.concatenate([Wq1, Wk1, Wv1, Ws1], axis=1)
    b1 = jnp.concatenate([bq1, bk1, bv1, bs1]).reshape(1, -1)
    w2 = jnp.concatenate([Wq2, Wk2, Wv2, Ws2], axis=1)
    b2 = jnp.concatenate([bq2, bk2, bv2, bs2]).reshape(1, -1)

    q1, k1, v1, sk1 = _proj1(xp, w1, b1)
    parts = _edge_softmax_aggregate(q1, k1, v1, srcp, dstp)
    q2, k2, v2, sk2 = _proj2(parts, sk1, w2, b2)
    parts = _edge_softmax_aggregate(q2, k2, v2, srcp, dstp)
    return _combine(parts, sk2)
